# baseline (speedup 1.0000x reference)
import functools

import jax
import jax.numpy as jnp
from jax import lax
from jax.experimental import pallas as pl
from jax.experimental.pallas import tpu as pltpu
from jax.experimental.pallas import tpu_sc as plsc

K_TOP = 8
TM = 1024
TT = 4096
BLK = 16


def _logits_block(x_ref, w_ref, bias_ref, o_ref):
    part = lax.dot_general(
        w_ref[...], x_ref[...],
        dimension_numbers=(((1,), (1,)), ((), ())),
        preferred_element_type=jnp.float32,
    )
    k = pl.program_id(1)

    @pl.when(k == 0)
    def _():
        o_ref[...] = part + bias_ref[...]

    @pl.when(k != 0)
    def _():
        o_ref[...] = o_ref[...] + part


def _transpose_block(wt_ref, o_ref):
    e = wt_ref.shape[0]
    r_i = lax.broadcasted_iota(jnp.int32, (e, e), 0)
    c_i = lax.broadcasted_iota(jnp.int32, (e, e), 1)
    ident = (r_i == c_i).astype(jnp.float32)
    o_ref[...] = lax.dot_general(
        wt_ref[...], ident,
        dimension_numbers=(((0,), (0,)), ((), ())),
        preferred_element_type=jnp.float32,
    )


def _ce_desc(a, b):
    return jnp.maximum(a, b), jnp.minimum(a, b)


_SORT8 = [(0, 1), (2, 3), (0, 2), (1, 3), (1, 2),
          (4, 5), (6, 7), (4, 6), (5, 7), (5, 6),
          (0, 4), (1, 5), (2, 6), (3, 7),
          (2, 4), (3, 5),
          (1, 2), (3, 4), (5, 6)]

_BITONIC8 = [(0, 4), (1, 5), (2, 6), (3, 7),
             (0, 2), (1, 3), (4, 6), (5, 7),
             (0, 1), (2, 3), (4, 5), (6, 7)]


def _sort8_desc(v):
    v = list(v)
    for i, j in _SORT8:
        v[i], v[j] = _ce_desc(v[i], v[j])
    return v


def _merge_top8_desc(a, b):
    c = [jnp.maximum(a[i], b[7 - i]) for i in range(8)]
    for i, j in _BITONIC8:
        c[i], c[j] = _ce_desc(c[i], c[j])
    return c


def _make_sc_route(n, e, rw):
    mesh = plsc.VectorSubcoreMesh(core_axis_name="c", subcore_axis_name="s")
    nblk = rw // BLK

    @functools.partial(
        pl.kernel,
        out_type=jax.ShapeDtypeStruct((e, n), jnp.float32),
        mesh=mesh,
        scratch_types=[
            pltpu.VMEM((e, rw), jnp.float32),
            pltpu.VMEM((e, rw), jnp.float32),
            pltpu.SemaphoreType.DMA,
            pltpu.SemaphoreType.DMA,
            pltpu.SemaphoreType.DMA,
        ],
    )
    def _route(lt_hbm, out_hbm, tbuf, obuf, sem_a, sem_b, sem_o):
        wid = lax.axis_index("s") * 2 + lax.axis_index("c")
        base = wid * rw
        half = rw // 2
        cp_a = pltpu.async_copy(
            lt_hbm.at[:, pl.ds(base, half)], tbuf.at[:, pl.ds(0, half)],
            sem_a)
        cp_b = pltpu.async_copy(
            lt_hbm.at[:, pl.ds(base + half, half)],
            tbuf.at[:, pl.ds(half, half)], sem_b)

        def blk_body(blk, carry):
            r0 = blk * BLK
            vals = [tbuf[j, pl.ds(r0, BLK)] for j in range(e)]
            one = jnp.full((BLK,), 1.0, jnp.float32)
            zero = jnp.zeros((BLK,), jnp.float32)
            tops = [_sort8_desc(vals[8 * g:8 * g + 8]) for g in range(e // 8)]
            while len(tops) > 2:
                tops = [_merge_top8_desc(tops[2 * g], tops[2 * g + 1])
                        for g in range(len(tops) // 2)]
            left, right = tops
            row_max = jnp.maximum(left[0], right[0])
            d = [jnp.maximum(left[i], right[7 - i]) for i in range(8)]
            t = d[0]
            for i in range(1, 8):
                t = jnp.minimum(t, d[i])
            ngt = zero
            for j in range(e):
                ngt = ngt + jnp.where(vals[j] > t, one, zero)
            keep = jnp.float32(K_TOP) - ngt
            run = zero
            denom = zero
            ps = []
            for j in range(e):
                pexp = jnp.exp(vals[j] - row_max)
                p_eq = jnp.where(run < keep, pexp, zero)
                p_eq = jnp.where(vals[j] == t, p_eq, zero)
                p = jnp.where(vals[j] > t, pexp, p_eq)
                run = run + jnp.where(vals[j] == t, one, zero)
                denom = denom + p
                ps.append(p)
            inv = one / denom
            for j in range(e):
                obuf[j, pl.ds(r0, BLK)] = ps[j] * inv
            return carry

        cp_a.wait()
        lax.fori_loop(0, nblk // 2, blk_body, 0)
        cp_o = pltpu.async_copy(
            obuf.at[:, pl.ds(0, half)], out_hbm.at[:, pl.ds(base, half)],
            sem_o)
        cp_b.wait()
        lax.fori_loop(nblk // 2, nblk, blk_body, 0)
        cp_o.wait()
        pltpu.sync_copy(
            obuf.at[:, pl.ds(half, half)],
            out_hbm.at[:, pl.ds(base + half, half)])

    return _route


@jax.jit
def _gate(x2d, W, bias_col):
    n, d = x2d.shape
    e = W.shape[0]
    logits_t = pl.pallas_call(
        _logits_block,
        grid=(n // TM, 2),
        in_specs=[
            pl.BlockSpec((TM, d // 2), lambda i, k: (i, k)),
            pl.BlockSpec((e, d // 2), lambda i, k: (0, k)),
            pl.BlockSpec((e, 1), lambda i, k: (0, 0)),
        ],
        out_specs=pl.BlockSpec((e, TM), lambda i, k: (0, i)),
        out_shape=jax.ShapeDtypeStruct((e, n), jnp.float32),
        compiler_params=pltpu.CompilerParams(
            dimension_semantics=("arbitrary", "arbitrary"),
        ),
    )(x2d, W, bias_col)
    info = plsc.get_sparse_core_info()
    nw = info.num_cores * info.num_subcores
    route = _make_sc_route(n, e, n // nw)
    weights_t = route(logits_t)
    return pl.pallas_call(
        _transpose_block,
        grid=(n // TT,),
        in_specs=[pl.BlockSpec((e, TT), lambda i: (0, i))],
        out_specs=pl.BlockSpec((TT, e), lambda i: (i, 0)),
        out_shape=jax.ShapeDtypeStruct((n, e), jnp.float32),
        compiler_params=pltpu.CompilerParams(
            dimension_semantics=("arbitrary",),
        ),
    )(weights_t)


def kernel(x, W, b, dynamic_bias):
    B, T, D = x.shape
    E = W.shape[0]
    x2d = x.reshape(B * T, D)
    bias_col = (b + dynamic_bias).reshape(E, 1)
    out = _gate(x2d, W, bias_col)
    return out.reshape(B, T, E)


# speedup vs baseline: 1.0244x; 1.0244x over previous
import functools

import jax
import jax.numpy as jnp
from jax import lax
from jax.experimental import pallas as pl
from jax.experimental.pallas import tpu as pltpu
from jax.experimental.pallas import tpu_sc as plsc

K_TOP = 8
TM = 1024
TT = 4096
BLK = 16


def _logits_block(x_ref, w_ref, bias_ref, o_ref):
    o_ref[...] = lax.dot_general(
        w_ref[...], x_ref[...],
        dimension_numbers=(((1,), (1,)), ((), ())),
        preferred_element_type=jnp.float32,
    ) + bias_ref[...]


def _transpose_block(wt_ref, o_ref):
    e = wt_ref.shape[0]
    r_i = lax.broadcasted_iota(jnp.int32, (e, e), 0)
    c_i = lax.broadcasted_iota(jnp.int32, (e, e), 1)
    ident = (r_i == c_i).astype(jnp.float32)
    o_ref[...] = lax.dot_general(
        wt_ref[...], ident,
        dimension_numbers=(((0,), (0,)), ((), ())),
        preferred_element_type=jnp.float32,
    )


def _ce_desc(a, b):
    return jnp.maximum(a, b), jnp.minimum(a, b)


_SORT8 = [(0, 1), (2, 3), (0, 2), (1, 3), (1, 2),
          (4, 5), (6, 7), (4, 6), (5, 7), (5, 6),
          (0, 4), (1, 5), (2, 6), (3, 7),
          (2, 4), (3, 5),
          (1, 2), (3, 4), (5, 6)]

_BITONIC8 = [(0, 4), (1, 5), (2, 6), (3, 7),
             (0, 2), (1, 3), (4, 6), (5, 7),
             (0, 1), (2, 3), (4, 5), (6, 7)]


def _sort8_desc(v):
    v = list(v)
    for i, j in _SORT8:
        v[i], v[j] = _ce_desc(v[i], v[j])
    return v


def _merge_top8_desc(a, b):
    c = [jnp.maximum(a[i], b[7 - i]) for i in range(8)]
    for i, j in _BITONIC8:
        c[i], c[j] = _ce_desc(c[i], c[j])
    return c


def _make_sc_route(n, e, rw):
    mesh = plsc.VectorSubcoreMesh(core_axis_name="c", subcore_axis_name="s")
    nblk = rw // BLK

    @functools.partial(
        pl.kernel,
        out_type=jax.ShapeDtypeStruct((e, n), jnp.float32),
        mesh=mesh,
        scratch_types=[
            pltpu.VMEM((e, rw), jnp.float32),
            pltpu.VMEM((e, rw), jnp.float32),
            pltpu.SemaphoreType.DMA,
            pltpu.SemaphoreType.DMA,
            pltpu.SemaphoreType.DMA,
        ],
    )
    def _route(lt_hbm, out_hbm, tbuf, obuf, sem_a, sem_b, sem_o):
        wid = lax.axis_index("s") * 2 + lax.axis_index("c")
        base = wid * rw
        half = rw // 2
        cp_a = pltpu.async_copy(
            lt_hbm.at[:, pl.ds(base, half)], tbuf.at[:, pl.ds(0, half)],
            sem_a)
        cp_b = pltpu.async_copy(
            lt_hbm.at[:, pl.ds(base + half, half)],
            tbuf.at[:, pl.ds(half, half)], sem_b)

        def blk_body(blk, carry):
            r0 = blk * BLK
            vals = [tbuf[j, pl.ds(r0, BLK)] for j in range(e)]
            one = jnp.full((BLK,), 1.0, jnp.float32)
            zero = jnp.zeros((BLK,), jnp.float32)
            tops = [_sort8_desc(vals[8 * g:8 * g + 8]) for g in range(e // 8)]
            while len(tops) > 2:
                tops = [_merge_top8_desc(tops[2 * g], tops[2 * g + 1])
                        for g in range(len(tops) // 2)]
            left, right = tops
            row_max = jnp.maximum(left[0], right[0])
            d = [jnp.maximum(left[i], right[7 - i]) for i in range(8)]
            t = d[0]
            for i in range(1, 8):
                t = jnp.minimum(t, d[i])
            ngt = zero
            for j in range(e):
                ngt = ngt + jnp.where(vals[j] > t, one, zero)
            keep = jnp.float32(K_TOP) - ngt
            run = zero
            denom = zero
            ps = []
            for j in range(e):
                pexp = jnp.exp(vals[j] - row_max)
                p_eq = jnp.where(run < keep, pexp, zero)
                p_eq = jnp.where(vals[j] == t, p_eq, zero)
                p = jnp.where(vals[j] > t, pexp, p_eq)
                run = run + jnp.where(vals[j] == t, one, zero)
                denom = denom + p
                ps.append(p)
            inv = one / denom
            for j in range(e):
                obuf[j, pl.ds(r0, BLK)] = ps[j] * inv
            return carry

        cp_a.wait()
        lax.fori_loop(0, nblk // 2, blk_body, 0)
        cp_o = pltpu.async_copy(
            obuf.at[:, pl.ds(0, half)], out_hbm.at[:, pl.ds(base, half)],
            sem_o)
        cp_b.wait()
        lax.fori_loop(nblk // 2, nblk, blk_body, 0)
        cp_o.wait()
        pltpu.sync_copy(
            obuf.at[:, pl.ds(half, half)],
            out_hbm.at[:, pl.ds(base + half, half)])

    return _route


@jax.jit
def _gate(x2d, W, bias_col):
    n, d = x2d.shape
    e = W.shape[0]
    logits_t = pl.pallas_call(
        _logits_block,
        grid=(n // TM,),
        in_specs=[
            pl.BlockSpec((TM, d), lambda i: (i, 0)),
            pl.BlockSpec((e, d), lambda i: (0, 0)),
            pl.BlockSpec((e, 1), lambda i: (0, 0)),
        ],
        out_specs=pl.BlockSpec((e, TM), lambda i: (0, i)),
        out_shape=jax.ShapeDtypeStruct((e, n), jnp.float32),
        compiler_params=pltpu.CompilerParams(
            dimension_semantics=("arbitrary",),
        ),
    )(x2d, W, bias_col)
    info = plsc.get_sparse_core_info()
    nw = info.num_cores * info.num_subcores
    route = _make_sc_route(n, e, n // nw)
    weights_t = route(logits_t)
    return pl.pallas_call(
        _transpose_block,
        grid=(n // TT,),
        in_specs=[pl.BlockSpec((e, TT), lambda i: (0, i))],
        out_specs=pl.BlockSpec((TT, e), lambda i: (i, 0)),
        out_shape=jax.ShapeDtypeStruct((n, e), jnp.float32),
        compiler_params=pltpu.CompilerParams(
            dimension_semantics=("arbitrary",),
        ),
    )(weights_t)


def kernel(x, W, b, dynamic_bias):
    B, T, D = x.shape
    E = W.shape[0]
    x2d = x.reshape(B * T, D)
    bias_col = (b + dynamic_bias).reshape(E, 1)
    out = _gate(x2d, W, bias_col)
    return out.reshape(B, T, E)


# speedup vs baseline: 1.0581x; 1.0329x over previous
import functools

import jax
import jax.numpy as jnp
from jax import lax
from jax.experimental import pallas as pl
from jax.experimental.pallas import tpu as pltpu
from jax.experimental.pallas import tpu_sc as plsc

K_TOP = 8
TM = 1024
TT = 4096
BLK = 16


def _logits_block(x_ref, w_ref, bias_ref, o_ref):
    o_ref[...] = lax.dot_general(
        w_ref[...], x_ref[...],
        dimension_numbers=(((1,), (1,)), ((), ())),
        preferred_element_type=jnp.float32,
    ) + bias_ref[...]


def _transpose_block(wt_ref, o_ref):
    e = wt_ref.shape[0]
    r_i = lax.broadcasted_iota(jnp.int32, (e, e), 0)
    c_i = lax.broadcasted_iota(jnp.int32, (e, e), 1)
    ident = (r_i == c_i).astype(jnp.float32)
    o_ref[...] = lax.dot_general(
        wt_ref[...], ident,
        dimension_numbers=(((0,), (0,)), ((), ())),
        preferred_element_type=jnp.float32,
    )


def _ce_desc(a, b):
    return jnp.maximum(a, b), jnp.minimum(a, b)


_SORT8 = [(0, 1), (2, 3), (0, 2), (1, 3), (1, 2),
          (4, 5), (6, 7), (4, 6), (5, 7), (5, 6),
          (0, 4), (1, 5), (2, 6), (3, 7),
          (2, 4), (3, 5),
          (1, 2), (3, 4), (5, 6)]

_BITONIC8 = [(0, 4), (1, 5), (2, 6), (3, 7),
             (0, 2), (1, 3), (4, 6), (5, 7),
             (0, 1), (2, 3), (4, 5), (6, 7)]


def _sort8_desc(v):
    v = list(v)
    for i, j in _SORT8:
        v[i], v[j] = _ce_desc(v[i], v[j])
    return v


def _merge_top8_desc(a, b):
    c = [jnp.maximum(a[i], b[7 - i]) for i in range(8)]
    for i, j in _BITONIC8:
        c[i], c[j] = _ce_desc(c[i], c[j])
    return c


def _make_sc_route(n, e, rw):
    mesh = plsc.VectorSubcoreMesh(core_axis_name="c", subcore_axis_name="s")
    nblk = rw // BLK

    @functools.partial(
        pl.kernel,
        out_type=jax.ShapeDtypeStruct((e, n), jnp.float32),
        mesh=mesh,
        scratch_types=[
            pltpu.VMEM((e, rw), jnp.float32),
            pltpu.VMEM((e, rw), jnp.float32),
            pltpu.SemaphoreType.DMA,
            pltpu.SemaphoreType.DMA,
            pltpu.SemaphoreType.DMA,
        ],
    )
    def _route(lt_hbm, out_hbm, tbuf, obuf, sem_a, sem_b, sem_o):
        wid = lax.axis_index("s") * 2 + lax.axis_index("c")
        base = wid * rw
        half = rw // 2
        cp_a = pltpu.async_copy(
            lt_hbm.at[:, pl.ds(base, half)], tbuf.at[:, pl.ds(0, half)],
            sem_a)
        cp_b = pltpu.async_copy(
            lt_hbm.at[:, pl.ds(base + half, half)],
            tbuf.at[:, pl.ds(half, half)], sem_b)

        def blk_body(blk, carry):
            r0 = blk * BLK
            vals = [tbuf[j, pl.ds(r0, BLK)] for j in range(e)]
            one = jnp.full((BLK,), 1.0, jnp.float32)
            zero = jnp.zeros((BLK,), jnp.float32)
            tops = [_sort8_desc(vals[8 * g:8 * g + 8]) for g in range(e // 8)]
            while len(tops) > 2:
                tops = [_merge_top8_desc(tops[2 * g], tops[2 * g + 1])
                        for g in range(len(tops) // 2)]
            left, right = tops
            row_max = jnp.maximum(left[0], right[0])
            d = [jnp.maximum(left[i], right[7 - i]) for i in range(8)]
            t = d[0]
            for i in range(1, 8):
                t = jnp.minimum(t, d[i])
            ngt = zero
            denom = zero
            for i in range(K_TOP):
                ngt = ngt + jnp.where(d[i] > t, one, zero)
                denom = denom + jnp.exp(d[i] - row_max)
            keep = jnp.float32(K_TOP) - ngt
            inv = one / denom
            run = zero
            for j in range(e):
                q = jnp.exp(vals[j] - row_max) * inv
                q_eq = jnp.where(run < keep, q, zero)
                q_eq = jnp.where(vals[j] == t, q_eq, zero)
                run = run + jnp.where(vals[j] == t, one, zero)
                obuf[j, pl.ds(r0, BLK)] = jnp.where(vals[j] > t, q, q_eq)
            return carry

        cp_a.wait()
        lax.fori_loop(0, nblk // 2, blk_body, 0)
        cp_o = pltpu.async_copy(
            obuf.at[:, pl.ds(0, half)], out_hbm.at[:, pl.ds(base, half)],
            sem_o)
        cp_b.wait()
        lax.fori_loop(nblk // 2, nblk, blk_body, 0)
        cp_o.wait()
        pltpu.sync_copy(
            obuf.at[:, pl.ds(half, half)],
            out_hbm.at[:, pl.ds(base + half, half)])

    return _route


@jax.jit
def _gate(x2d, W, bias_col):
    n, d = x2d.shape
    e = W.shape[0]
    logits_t = pl.pallas_call(
        _logits_block,
        grid=(n // TM,),
        in_specs=[
            pl.BlockSpec((TM, d), lambda i: (i, 0)),
            pl.BlockSpec((e, d), lambda i: (0, 0)),
            pl.BlockSpec((e, 1), lambda i: (0, 0)),
        ],
        out_specs=pl.BlockSpec((e, TM), lambda i: (0, i)),
        out_shape=jax.ShapeDtypeStruct((e, n), jnp.float32),
        compiler_params=pltpu.CompilerParams(
            dimension_semantics=("arbitrary",),
        ),
    )(x2d, W, bias_col)
    info = plsc.get_sparse_core_info()
    nw = info.num_cores * info.num_subcores
    route = _make_sc_route(n, e, n // nw)
    weights_t = route(logits_t)
    return pl.pallas_call(
        _transpose_block,
        grid=(n // TT,),
        in_specs=[pl.BlockSpec((e, TT), lambda i: (0, i))],
        out_specs=pl.BlockSpec((TT, e), lambda i: (i, 0)),
        out_shape=jax.ShapeDtypeStruct((n, e), jnp.float32),
        compiler_params=pltpu.CompilerParams(
            dimension_semantics=("arbitrary",),
        ),
    )(weights_t)


def kernel(x, W, b, dynamic_bias):
    B, T, D = x.shape
    E = W.shape[0]
    x2d = x.reshape(B * T, D)
    bias_col = (b + dynamic_bias).reshape(E, 1)
    out = _gate(x2d, W, bias_col)
    return out.reshape(B, T, E)


# speedup vs baseline: 1.1186x; 1.0572x over previous
import functools

import jax
import jax.numpy as jnp
from jax import lax
from jax.experimental import pallas as pl
from jax.experimental.pallas import tpu as pltpu
from jax.experimental.pallas import tpu_sc as plsc

K_TOP = 8
TM = 1024
TT = 4096
BLK = 16


def _logits_block(x_ref, w_ref, bias_ref, o_ref):
    o_ref[...] = lax.dot_general(
        w_ref[...], x_ref[...],
        dimension_numbers=(((1,), (1,)), ((), ())),
        preferred_element_type=jnp.float32,
    ) + bias_ref[...]


def _transpose_block(wt_ref, o_ref):
    e = wt_ref.shape[0]
    r_i = lax.broadcasted_iota(jnp.int32, (e, e), 0)
    c_i = lax.broadcasted_iota(jnp.int32, (e, e), 1)
    ident = (r_i == c_i).astype(jnp.float32)
    o_ref[...] = lax.dot_general(
        wt_ref[...], ident,
        dimension_numbers=(((0,), (0,)), ((), ())),
        preferred_element_type=jnp.float32,
    )


def _ce_desc(a, b):
    return jnp.maximum(a, b), jnp.minimum(a, b)


_SORT8 = [(0, 1), (2, 3), (0, 2), (1, 3), (1, 2),
          (4, 5), (6, 7), (4, 6), (5, 7), (5, 6),
          (0, 4), (1, 5), (2, 6), (3, 7),
          (2, 4), (3, 5),
          (1, 2), (3, 4), (5, 6)]

_BITONIC8 = [(0, 4), (1, 5), (2, 6), (3, 7),
             (0, 2), (1, 3), (4, 6), (5, 7),
             (0, 1), (2, 3), (4, 5), (6, 7)]


def _sort8_desc(v):
    v = list(v)
    for i, j in _SORT8:
        v[i], v[j] = _ce_desc(v[i], v[j])
    return v


def _merge_top8_desc(a, b):
    c = [jnp.maximum(a[i], b[7 - i]) for i in range(8)]
    for i, j in _BITONIC8:
        c[i], c[j] = _ce_desc(c[i], c[j])
    return c


def _make_sc_route(n, e, rw):
    mesh = plsc.VectorSubcoreMesh(core_axis_name="c", subcore_axis_name="s")
    nblk = rw // BLK

    @functools.partial(
        pl.kernel,
        out_type=jax.ShapeDtypeStruct((e, n), jnp.float32),
        mesh=mesh,
        scratch_types=[
            pltpu.VMEM((e, rw), jnp.float32),
            pltpu.VMEM((e, rw), jnp.float32),
            pltpu.SemaphoreType.DMA,
            pltpu.SemaphoreType.DMA,
            pltpu.SemaphoreType.DMA,
        ],
    )
    def _route(lt_hbm, out_hbm, tbuf, obuf, sem_a, sem_b, sem_o):
        wid = lax.axis_index("s") * 2 + lax.axis_index("c")
        base = wid * rw
        half = rw // 2
        cp_a = pltpu.async_copy(
            lt_hbm.at[:, pl.ds(base, half)], tbuf.at[:, pl.ds(0, half)],
            sem_a)
        cp_b = pltpu.async_copy(
            lt_hbm.at[:, pl.ds(base + half, half)],
            tbuf.at[:, pl.ds(half, half)], sem_b)

        def blk_body(blk, carry):
            r0 = blk * BLK
            vals = [tbuf[j, pl.ds(r0, BLK)] for j in range(e)]
            one = jnp.full((BLK,), 1.0, jnp.float32)
            zero = jnp.zeros((BLK,), jnp.float32)
            tops = [_sort8_desc(vals[8 * g:8 * g + 8]) for g in range(e // 8)]
            while len(tops) > 2:
                tops = [_merge_top8_desc(tops[2 * g], tops[2 * g + 1])
                        for g in range(len(tops) // 2)]
            left, right = tops
            row_max = jnp.maximum(left[0], right[0])
            d = [jnp.maximum(left[i], right[7 - i]) for i in range(8)]
            t = d[0]
            for i in range(1, 8):
                t = jnp.minimum(t, d[i])
            ngt = zero
            denom = zero
            for i in range(K_TOP):
                ngt = ngt + jnp.where(d[i] > t, one, zero)
                denom = denom + jnp.exp(d[i] - row_max)
            keep = jnp.float32(K_TOP) - ngt
            inv = one / denom
            run = zero
            for j in range(e):
                q = jnp.exp(vals[j] - row_max) * inv
                q_eq = jnp.where(run < keep, q, zero)
                q_eq = jnp.where(vals[j] == t, q_eq, zero)
                run = run + jnp.where(vals[j] == t, one, zero)
                obuf[j, pl.ds(r0, BLK)] = jnp.where(vals[j] > t, q, q_eq)
            return carry

        cp_a.wait()
        lax.fori_loop(0, nblk // 2, blk_body, 0)
        cp_o = pltpu.async_copy(
            obuf.at[:, pl.ds(0, half)], out_hbm.at[:, pl.ds(base, half)],
            sem_o)
        cp_b.wait()
        lax.fori_loop(nblk // 2, nblk, blk_body, 0)
        cp_o.wait()
        pltpu.sync_copy(
            obuf.at[:, pl.ds(half, half)],
            out_hbm.at[:, pl.ds(base + half, half)])

    return _route


@jax.jit
def _gate(x2d, W, bias_col):
    n, d = x2d.shape
    e = W.shape[0]
    logits_t = pl.pallas_call(
        _logits_block,
        grid=(n // TM,),
        in_specs=[
            pl.BlockSpec((TM, d), lambda i: (i, 0)),
            pl.BlockSpec((e, d), lambda i: (0, 0)),
            pl.BlockSpec((e, 1), lambda i: (0, 0)),
        ],
        out_specs=pl.BlockSpec((e, TM), lambda i: (0, i)),
        out_shape=jax.ShapeDtypeStruct((e, n), jnp.float32),
        compiler_params=pltpu.CompilerParams(
            dimension_semantics=("arbitrary",),
        ),
    )(x2d, W, bias_col)
    info = plsc.get_sparse_core_info()
    nw = info.num_cores * info.num_subcores
    route = _make_sc_route(n, e, n // nw)
    weights_t = route(logits_t)
    return weights_t.T


def kernel(x, W, b, dynamic_bias):
    B, T, D = x.shape
    E = W.shape[0]
    x2d = x.reshape(B * T, D)
    bias_col = (b + dynamic_bias).reshape(E, 1)
    out = _gate(x2d, W, bias_col)
    return out.reshape(B, T, E)


# speedup vs baseline: 1.1435x; 1.0222x over previous
import functools

import jax
import jax.numpy as jnp
from jax import lax
from jax.experimental import pallas as pl
from jax.experimental.pallas import tpu as pltpu
from jax.experimental.pallas import tpu_sc as plsc

K_TOP = 8
TM = 1024
BLK = 16


def _logits_block(x_ref, w_ref, bias_ref, o_ref):
    o_ref[...] = lax.dot_general(
        w_ref[...], x_ref[...],
        dimension_numbers=(((1,), (1,)), ((), ())),
        preferred_element_type=jnp.float32,
    ) + bias_ref[...]


def _ce_desc(a, b):
    return jnp.maximum(a, b), jnp.minimum(a, b)


_SORT8 = [(0, 1), (2, 3), (0, 2), (1, 3), (1, 2),
          (4, 5), (6, 7), (4, 6), (5, 7), (5, 6),
          (0, 4), (1, 5), (2, 6), (3, 7),
          (2, 4), (3, 5),
          (1, 2), (3, 4), (5, 6)]

_BITONIC8 = [(0, 4), (1, 5), (2, 6), (3, 7),
             (0, 2), (1, 3), (4, 6), (5, 7),
             (0, 1), (2, 3), (4, 5), (6, 7)]


def _sort8_desc(v):
    v = list(v)
    for i, j in _SORT8:
        v[i], v[j] = _ce_desc(v[i], v[j])
    return v


def _merge_top8_desc(a, b):
    c = [jnp.maximum(a[i], b[7 - i]) for i in range(8)]
    for i, j in _BITONIC8:
        c[i], c[j] = _ce_desc(c[i], c[j])
    return c


def _make_sc_route(n, e, rw):
    mesh = plsc.VectorSubcoreMesh(core_axis_name="c", subcore_axis_name="s")
    nblk = rw // BLK

    @functools.partial(
        pl.kernel,
        out_type=jax.ShapeDtypeStruct((e, n), jnp.float32),
        mesh=mesh,
        scratch_types=[
            pltpu.VMEM((e, rw), jnp.float32),
            pltpu.VMEM((e, rw), jnp.float32),
            pltpu.SemaphoreType.DMA,
            pltpu.SemaphoreType.DMA,
            pltpu.SemaphoreType.DMA,
        ],
    )
    def _route(lt_hbm, out_hbm, tbuf, obuf, sem_a, sem_b, sem_o):
        wid = lax.axis_index("s") * 2 + lax.axis_index("c")
        base = wid * rw
        half = rw // 2
        cp_a = pltpu.async_copy(
            lt_hbm.at[:, pl.ds(base, half)], tbuf.at[:, pl.ds(0, half)],
            sem_a)
        cp_b = pltpu.async_copy(
            lt_hbm.at[:, pl.ds(base + half, half)],
            tbuf.at[:, pl.ds(half, half)], sem_b)

        def blk_body(blk, carry):
            r0 = blk * BLK
            vals = [tbuf[j, pl.ds(r0, BLK)] for j in range(e)]
            one = jnp.full((BLK,), 1.0, jnp.float32)
            zero = jnp.zeros((BLK,), jnp.float32)
            tops = [_sort8_desc(vals[8 * g:8 * g + 8]) for g in range(e // 8)]
            while len(tops) > 2:
                tops = [_merge_top8_desc(tops[2 * g], tops[2 * g + 1])
                        for g in range(len(tops) // 2)]
            left, right = tops
            row_max = jnp.maximum(left[0], right[0])
            d = [jnp.maximum(left[i], right[7 - i]) for i in range(8)]
            t = d[0]
            for i in range(1, 8):
                t = jnp.minimum(t, d[i])
            ngt = zero
            denom = zero
            for i in range(K_TOP):
                ngt = ngt + jnp.where(d[i] > t, one, zero)
                denom = denom + jnp.exp(d[i] - row_max)
            keep = jnp.float32(K_TOP) - ngt
            inv = one / denom
            run = zero
            for j in range(e):
                q = jnp.exp(vals[j] - row_max) * inv
                q_eq = jnp.where(run < keep, q, zero)
                q_eq = jnp.where(vals[j] == t, q_eq, zero)
                run = run + jnp.where(vals[j] == t, one, zero)
                obuf[j, pl.ds(r0, BLK)] = jnp.where(vals[j] > t, q, q_eq)
            return carry

        cp_a.wait()
        lax.fori_loop(0, nblk // 2, blk_body, 0)
        cp_o = pltpu.async_copy(
            obuf.at[:, pl.ds(0, half)], out_hbm.at[:, pl.ds(base, half)],
            sem_o)
        cp_b.wait()
        lax.fori_loop(nblk // 2, nblk, blk_body, 0)
        cp_o.wait()
        pltpu.sync_copy(
            obuf.at[:, pl.ds(half, half)],
            out_hbm.at[:, pl.ds(base + half, half)])

    return _route


@jax.jit
def _gate(x2d, W, bias_col):
    n, d = x2d.shape
    e = W.shape[0]
    logits_t = pl.pallas_call(
        _logits_block,
        grid=(n // TM,),
        in_specs=[
            pl.BlockSpec((TM, d), lambda i: (i, 0)),
            pl.BlockSpec((e, d), lambda i: (0, 0)),
            pl.BlockSpec((e, 1), lambda i: (0, 0)),
        ],
        out_specs=pl.BlockSpec((e, TM), lambda i: (0, i)),
        out_shape=jax.ShapeDtypeStruct((e, n), jnp.float32),
        compiler_params=pltpu.CompilerParams(
            dimension_semantics=("arbitrary",),
        ),
    )(x2d, W, bias_col)
    info = plsc.get_sparse_core_info()
    nw = info.num_cores * info.num_subcores
    route = _make_sc_route(n, e, n // nw)
    weights_t = route(logits_t)
    return weights_t.T


def kernel(x, W, b, dynamic_bias):
    B, T, D = x.shape
    E = W.shape[0]
    x2d = x.reshape(B * T, D)
    bias_col = (b + dynamic_bias).reshape(E, 1)
    out = _gate(x2d, W, bias_col)
    return out.reshape(B, T, E)
